# baseline (speedup 1.0000x reference)
import functools

import jax
import jax.numpy as jnp
from jax import lax
from jax.experimental import pallas as pl
from jax.experimental.pallas import tpu as pltpu

N_TOK = 128
D_IN = 1024
V_DICT = 65536
TOPK = 64

ENC_BLK = 4096
DEC_BLK = 4096


def _encode_body(x_ref, w_ref, b_ref, out_ref):
    acts = lax.dot_general(
        x_ref[...], w_ref[...],
        dimension_numbers=(((1,), (1,)), ((), ())),
        preferred_element_type=jnp.float32,
    )
    out_ref[...] = jnp.maximum(acts + b_ref[...], 0.0)


def _encode(xc, enc_w, enc_b2d):
    grid = (V_DICT // ENC_BLK,)
    return pl.pallas_call(
        _encode_body,
        grid=grid,
        in_specs=[
            pl.BlockSpec((N_TOK, D_IN), lambda i: (0, 0)),
            pl.BlockSpec((ENC_BLK, D_IN), lambda i: (i, 0)),
            pl.BlockSpec((1, ENC_BLK), lambda i: (0, i)),
        ],
        out_specs=pl.BlockSpec((N_TOK, ENC_BLK), lambda i: (0, i)),
        out_shape=jax.ShapeDtypeStruct((N_TOK, V_DICT), jnp.float32),
    )(xc, enc_w, enc_b2d)


def _decode_body(sc_ref, w_ref, b_ref, out_ref):
    @pl.when(pl.program_id(0) == 0)
    def _():
        out_ref[...] = jnp.broadcast_to(b_ref[...], (N_TOK, D_IN))

    out_ref[...] += lax.dot_general(
        sc_ref[...], w_ref[...],
        dimension_numbers=(((1,), (1,)), ((), ())),
        preferred_element_type=jnp.float32,
    )


def _decode(sparse_code, dec_w, dec_bias2d):
    grid = (V_DICT // DEC_BLK,)
    return pl.pallas_call(
        _decode_body,
        grid=grid,
        in_specs=[
            pl.BlockSpec((N_TOK, DEC_BLK), lambda i: (0, i)),
            pl.BlockSpec((D_IN, DEC_BLK), lambda i: (0, i)),
            pl.BlockSpec((1, D_IN), lambda i: (0, 0)),
        ],
        out_specs=pl.BlockSpec((N_TOK, D_IN), lambda i: (0, 0)),
        out_shape=jax.ShapeDtypeStruct((N_TOK, D_IN), jnp.float32),
    )(sparse_code, dec_w, dec_bias2d)


@jax.jit
def kernel(x, enc_w, enc_b, dec_w, dec_bias):
    xc = x - dec_bias
    acts = _encode(xc, enc_w, enc_b.reshape(1, V_DICT))
    top_v, top_i = lax.top_k(acts, TOPK)
    rows = jnp.arange(N_TOK)[:, None]
    sparse_code = jnp.zeros_like(acts).at[rows, top_i].set(top_v)
    recon = _decode(sparse_code, dec_w, dec_bias.reshape(1, D_IN))
    return (recon, sparse_code)


# speedup vs baseline: 4.6648x; 4.6648x over previous
import functools

import jax
import jax.numpy as jnp
from jax import lax
from jax.experimental import pallas as pl
from jax.experimental.pallas import tpu as pltpu
import jax.experimental.pallas.tpu_sc as plsc

N_TOK = 128
D_IN = 1024
V_DICT = 65536
TOPK = 64

ENC_BLK = 4096
DEC_BLK = 4096

_LANES = 16
_NVREG = V_DICT // _LANES
_L1 = _NVREG // 8
_L2 = _L1 // 8
_NWORK = 32
_RPW = N_TOK // _NWORK
_HITS_CAP = 1024
_CAND_CAP = 4096
_ZBUF = 16384


def _encode_body(x_ref, w_ref, b_ref, out_ref):
    acts = lax.dot_general(
        x_ref[...], w_ref[...],
        dimension_numbers=(((1,), (1,)), ((), ())),
        preferred_element_type=jnp.float32,
    )
    out_ref[...] = jnp.maximum(acts + b_ref[...], 0.0)


def _encode(xc, enc_w, enc_b2d):
    return pl.pallas_call(
        _encode_body,
        grid=(V_DICT // ENC_BLK,),
        in_specs=[
            pl.BlockSpec((N_TOK, D_IN), lambda i: (0, 0)),
            pl.BlockSpec((ENC_BLK, D_IN), lambda i: (i, 0)),
            pl.BlockSpec((1, ENC_BLK), lambda i: (0, i)),
        ],
        out_specs=pl.BlockSpec((N_TOK, ENC_BLK), lambda i: (0, i)),
        out_shape=jax.ShapeDtypeStruct((N_TOK, V_DICT), jnp.float32),
    )(xc, enc_w, enc_b2d)


def _decode_body(sc_ref, w_ref, b_ref, out_ref):
    @pl.when(pl.program_id(0) == 0)
    def _():
        out_ref[...] = jnp.broadcast_to(b_ref[...], (N_TOK, D_IN))

    out_ref[...] += lax.dot_general(
        sc_ref[...], w_ref[...],
        dimension_numbers=(((1,), (1,)), ((), ())),
        preferred_element_type=jnp.float32,
    )


def _decode(sparse_code, dec_w, dec_bias2d):
    return pl.pallas_call(
        _decode_body,
        grid=(V_DICT // DEC_BLK,),
        in_specs=[
            pl.BlockSpec((N_TOK, DEC_BLK), lambda i: (0, i)),
            pl.BlockSpec((D_IN, DEC_BLK), lambda i: (0, i)),
            pl.BlockSpec((1, D_IN), lambda i: (0, 0)),
        ],
        out_specs=pl.BlockSpec((N_TOK, D_IN), lambda i: (0, 0)),
        out_shape=jax.ShapeDtypeStruct((N_TOK, D_IN), jnp.float32),
    )(sparse_code, dec_w, dec_bias2d)


def _allreduce(v, op):
    iota = jnp.arange(_LANES, dtype=jnp.int32)
    for sh in (8, 4, 2, 1):
        v = op(v, v.at[iota ^ sh].get(mode="promise_in_bounds"))
    return v


def _sc_select_body(acts_hbm, sparse_hbm, row_v, l1_v, l2_v, hits_v,
                    cand_v, cand_i, topv_v, topi_v, zero_v, zsem, ssem):
    cid = lax.axis_index("c")
    sid = lax.axis_index("s")
    wid = sid * 2 + cid
    iota = jnp.arange(_LANES, dtype=jnp.int32)
    zvec = jnp.zeros((_LANES,), jnp.float32)

    def _zero_init(i, carry):
        zero_v[pl.ds(i * _LANES, _LANES)] = zvec
        return carry

    lax.fori_loop(0, _ZBUF // _LANES, _zero_init, 0)

    def _row(rho, carry):
        r = wid * _RPW + rho

        zcopies = [
            pltpu.async_copy(
                zero_v, sparse_hbm.at[pl.ds(r * V_DICT + q * _ZBUF, _ZBUF)],
                zsem)
            for q in range(V_DICT // _ZBUF)
        ]

        pltpu.sync_copy(acts_hbm.at[r], row_v)

        def _l1(c, carry):
            base = c * 128
            acc = row_v[pl.ds(base, _LANES)]
            for j in range(1, 8):
                acc = jnp.maximum(acc, row_v[pl.ds(base + j * _LANES, _LANES)])
            l1_v[pl.ds(c * _LANES, _LANES)] = acc
            return carry

        lax.fori_loop(0, _L1, _l1, 0)

        def _l2(g, carry):
            base = g * 128
            acc = l1_v[pl.ds(base, _LANES)]
            for j in range(1, 8):
                acc = jnp.maximum(acc, l1_v[pl.ds(base + j * _LANES, _LANES)])
            l2_v[pl.ds(g * _LANES, _LANES)] = acc
            return carry

        lax.fori_loop(0, _L2, _l2, 0)

        def _count_ge(t_f):
            tb = jnp.full((_LANES,), t_f, jnp.float32)

            def _cnt(g, acc):
                v = l2_v[pl.ds(g * _LANES, _LANES)]
                return acc + jnp.where(v >= tb, 1, 0).astype(jnp.int32)

            acc = lax.fori_loop(0, _L2, _cnt, jnp.zeros((_LANES,), jnp.int32))
            return _allreduce(acc, jnp.add)[0]

        def _bsearch(i, lohi):
            lo, hi = lohi
            mid = lo + ((hi - lo + 1) >> 1)
            mid_f = lax.bitcast_convert_type(mid, jnp.float32)
            feas = _count_ge(mid_f) >= TOPK
            return (jnp.where(feas, mid, lo), jnp.where(feas, hi, mid - 1))

        lo0 = jnp.int32(0)
        hi0 = jnp.int32(0x7F800000)
        lo, hi = lax.fori_loop(0, 31, _bsearch, (lo0, hi0))
        tau_s = lax.bitcast_convert_type(lo, jnp.float32)
        tau = jnp.full((_LANES,), tau_s, jnp.float32)

        def _hits(c, off):
            v = l1_v[pl.ds(c * _LANES, _LANES)]
            m = v >= tau
            offc = jnp.minimum(off, _HITS_CAP)
            plsc.store_compressed(hits_v.at[pl.ds(offc, _LANES)],
                                  c * _LANES + iota, mask=m)
            return off + plsc.all_reduce_population_count(m)[0]

        n_hits = lax.fori_loop(0, _L1, _hits, jnp.int32(0))
        n_hits = jnp.minimum(n_hits, _HITS_CAP)

        k_lo = iota & 7
        is_hi = iota >= 8

        def _gather(p, off):
            h2 = hits_v[pl.ds(2 * p, _LANES)]
            g0 = jnp.full((_LANES,), h2[0], jnp.int32)
            g1 = jnp.full((_LANES,), h2[1], jnp.int32)
            g = jnp.where(is_hi, g1, g0)
            valid = jnp.logical_or(jnp.logical_not(is_hi), 2 * p + 1 < n_hits)
            eidx = (((g >> 4) << 7) + (g & 15) + (k_lo << 4)) & (V_DICT - 1)
            vals = plsc.load_gather(row_v, [eidx])
            m = jnp.logical_and(jnp.logical_and(vals >= tau, valid),
                                off < _CAND_CAP)
            offc = jnp.minimum(off, _CAND_CAP)
            plsc.store_compressed(cand_v.at[pl.ds(offc, _LANES)], vals, mask=m)
            plsc.store_compressed(cand_i.at[pl.ds(offc, _LANES)], eidx, mask=m)
            return off + plsc.all_reduce_population_count(m)[0]

        npairs = lax.div(n_hits + 1, jnp.int32(2))
        n_cand = lax.fori_loop(0, npairs, _gather, jnp.int32(0))
        n_cand = jnp.minimum(n_cand, _CAND_CAP)

        cand_v[pl.ds(n_cand, _LANES)] = jnp.full((_LANES,), -1.0, jnp.float32)
        cand_i[pl.ds(n_cand, _LANES)] = jnp.full((_LANES,), 0x3FFFFFFF,
                                                 jnp.int32)
        nv = lax.div(n_cand + 15, jnp.int32(_LANES))

        def _extract(t, carry):
            def _scan(m, acc):
                bv, bi, bp = acc
                v = cand_v[pl.ds(m * _LANES, _LANES)]
                vi = cand_i[pl.ds(m * _LANES, _LANES)]
                upd = jnp.logical_or(
                    v > bv, jnp.logical_and(v == bv, vi < bi))
                return (jnp.where(upd, v, bv), jnp.where(upd, vi, bi),
                        jnp.where(upd, m * _LANES + iota, bp))

            bv0 = jnp.full((_LANES,), -3e38, jnp.float32)
            bi0 = jnp.full((_LANES,), 0x7FFFFFFF, jnp.int32)
            bp0 = jnp.zeros((_LANES,), jnp.int32)
            bv, bi, bp = lax.fori_loop(0, nv, _scan, (bv0, bi0, bp0))
            mv = _allreduce(bv, jnp.maximum)
            lm = bv == mv
            big = jnp.full((_LANES,), 0x7FFFFFFF, jnp.int32)
            li = _allreduce(jnp.where(lm, bi, big), jnp.minimum)
            pm = jnp.logical_and(lm, bi == li)
            pos = _allreduce(jnp.where(pm, bp, big), jnp.minimum)

            lane0 = iota == 0
            tsplat = jnp.full((_LANES,), t, jnp.int32)
            plsc.store_scatter(topv_v, [tsplat], mv, mask=lane0)
            plsc.store_scatter(topi_v, [tsplat], r * V_DICT + li, mask=lane0)
            plsc.store_scatter(cand_v, [pos],
                               jnp.full((_LANES,), -2.0, jnp.float32),
                               mask=lane0)
            return carry

        lax.fori_loop(0, TOPK, _extract, 0)

        for zc in zcopies:
            zc.wait()
        pltpu.async_copy(topv_v, sparse_hbm.at[topi_v], ssem).wait()
        return carry

    lax.fori_loop(0, _RPW, _row, 0)


def _make_sc_select(interpret=False):
    return pl.kernel(
        _sc_select_body,
        out_type=jax.ShapeDtypeStruct((N_TOK * V_DICT,), jnp.float32),
        mesh=plsc.VectorSubcoreMesh(core_axis_name="c", subcore_axis_name="s",
                                    num_cores=2, num_subcores=16),
        compiler_params=pltpu.CompilerParams(needs_layout_passes=False),
        scratch_types=[
            pltpu.VMEM((V_DICT,), jnp.float32),
            pltpu.VMEM((_L1 * _LANES,), jnp.float32),
            pltpu.VMEM((_L2 * _LANES,), jnp.float32),
            pltpu.VMEM((_HITS_CAP + 2 * _LANES,), jnp.int32),
            pltpu.VMEM((_CAND_CAP + 2 * _LANES,), jnp.float32),
            pltpu.VMEM((_CAND_CAP + 2 * _LANES,), jnp.int32),
            pltpu.VMEM((TOPK,), jnp.float32),
            pltpu.VMEM((TOPK,), jnp.int32),
            pltpu.VMEM((_ZBUF,), jnp.float32),
            pltpu.SemaphoreType.DMA,
            pltpu.SemaphoreType.DMA,
        ],
        interpret=interpret,
    )


_sc_select = _make_sc_select()


@jax.jit
def kernel(x, enc_w, enc_b, dec_w, dec_bias):
    xc = x - dec_bias
    acts = _encode(xc, enc_w, enc_b.reshape(1, V_DICT))
    sparse_flat = _sc_select(acts)
    sparse_code = sparse_flat.reshape(N_TOK, V_DICT)
    recon = _decode(sparse_code, dec_w, dec_bias.reshape(1, D_IN))
    return (recon, sparse_code)
